# Initial kernel scaffold; baseline (speedup 1.0000x reference)
import jax
import jax.numpy as jnp
from jax.experimental import pallas as pl


def kernel(x, vectors):
    raise NotImplementedError("write your pallas kernel here")


# speedup vs baseline: 1.4033x; 1.4033x over previous
import functools

import jax
import jax.numpy as jnp
from jax import lax
from jax.experimental import pallas as pl
from jax.experimental.pallas import tpu as pltpu, tpu_sc as plsc

N = 32768
D = 256
K = 8192
RB = 256
NRB = N // RB

NC = 2
NS = 16
NW = NC * NS
BPW = N // NW
CH = 128
NCHUNK = BPW // CH


def _argmin_body(x_ref, v_ref, idx_ref, loss_ref, v2_ref, acc_ref):
    i = pl.program_id(0)
    v = v_ref[...]

    @pl.when(i == 0)
    def _():
        v2_ref[...] = jnp.sum(v * v, axis=0, keepdims=True)

    x = x_ref[...]
    x2 = jnp.sum(x * x, axis=1, keepdims=True)
    xv = jnp.dot(x, v, preferred_element_type=jnp.float32)
    scores = (x2 - 2.0 * xv) + v2_ref[...]
    rowmin = jnp.min(scores, axis=1, keepdims=True)
    cols = lax.broadcasted_iota(jnp.int32, (RB, K), 1)
    masked = jnp.where(scores == rowmin, cols, jnp.int32(K))
    idx_ref[...] = jnp.min(masked, axis=1, keepdims=True)

    part = jnp.sum(rowmin, axis=0, keepdims=True)

    @pl.when(i == 0)
    def _():
        acc_ref[...] = part

    @pl.when(i > 0)
    def _():
        acc_ref[...] = acc_ref[...] + part

    @pl.when(i == NRB - 1)
    def _():
        loss_ref[...] = acc_ref[...] / (N * D)


_argmin_call = pl.pallas_call(
    _argmin_body,
    grid=(NRB,),
    in_specs=[
        pl.BlockSpec((RB, D), lambda i: (i, 0)),
        pl.BlockSpec((D, K), lambda i: (0, 0)),
    ],
    out_specs=[
        pl.BlockSpec((RB, 1), lambda i: (i, 0)),
        pl.BlockSpec((1, 1), lambda i: (0, 0)),
    ],
    out_shape=[
        jax.ShapeDtypeStruct((N, 1), jnp.int32),
        jax.ShapeDtypeStruct((1, 1), jnp.float32),
    ],
    scratch_shapes=[
        pltpu.VMEM((1, K), jnp.float32),
        pltpu.VMEM((1, 1), jnp.float32),
    ],
    compiler_params=pltpu.CompilerParams(
        dimension_semantics=("arbitrary",),
    ),
)


@functools.cache
def _make_gather():
    @functools.partial(
        pl.kernel,
        mesh=plsc.VectorSubcoreMesh(
            core_axis_name="c", subcore_axis_name="s", num_cores=NC
        ),
        out_type=jax.ShapeDtypeStruct((N, D), jnp.float32),
        scratch_types=[
            pltpu.VMEM((CH,), jnp.int32),
            pltpu.VMEM((CH, D), jnp.float32),
            pltpu.SemaphoreType.DMA,
        ],
    )
    def _gather(table_hbm, idx_hbm, out_hbm, idx_v, rows_v, sem):
        wid = lax.axis_index("s") * NC + lax.axis_index("c")
        for c in range(NCHUNK):
            base = wid * BPW + c * CH
            pltpu.sync_copy(idx_hbm.at[pl.ds(base, CH)], idx_v)
            pltpu.async_copy(table_hbm.at[idx_v], rows_v, sem).wait()
            pltpu.sync_copy(rows_v, out_hbm.at[pl.ds(base, CH)])

    return _gather


def kernel(x, vectors):
    idx2d, loss = _argmin_call(x, vectors)
    quantized = _make_gather()(vectors.T, idx2d.reshape(-1))
    l = loss[0, 0]
    return quantized, l, l, idx2d


# speedup vs baseline: 1.4973x; 1.0670x over previous
import functools

import jax
import jax.numpy as jnp
from jax import lax
from jax.experimental import pallas as pl
from jax.experimental.pallas import tpu as pltpu, tpu_sc as plsc

N = 32768
D = 256
K = 8192
RB = 256
NRB = N // RB

NC = 2
NS = 16
NW = NC * NS
BPW = N // NW
CH = 128
NCHUNK = BPW // CH


def _argmin_body(x_ref, v_ref, idx_ref, loss_ref, v2_ref, cols_ref, acc_ref):
    i = pl.program_id(0)
    v = v_ref[...]

    @pl.when(i == 0)
    def _():
        v2_ref[...] = jnp.sum(v * v, axis=0, keepdims=True)
        cols_ref[...] = lax.broadcasted_iota(jnp.int32, (1, K), 1).astype(
            jnp.float32
        )

    x = x_ref[...]
    x2 = jnp.sum(x * x, axis=1, keepdims=True)
    xv2 = jnp.dot(x * -2.0, v, preferred_element_type=jnp.float32)
    scores = (x2 + xv2) + v2_ref[...]
    rowmin = jnp.min(scores, axis=1, keepdims=True)
    masked = jnp.where(scores == rowmin, cols_ref[...], jnp.float32(K))
    idx_ref[...] = jnp.min(masked, axis=1, keepdims=True).astype(jnp.int32)

    part = jnp.sum(rowmin, axis=0, keepdims=True)

    @pl.when(i == 0)
    def _():
        acc_ref[...] = part

    @pl.when(i > 0)
    def _():
        acc_ref[...] = acc_ref[...] + part

    @pl.when(i == NRB - 1)
    def _():
        loss_ref[...] = acc_ref[...] / (N * D)


_argmin_call = pl.pallas_call(
    _argmin_body,
    grid=(NRB,),
    in_specs=[
        pl.BlockSpec((RB, D), lambda i: (i, 0)),
        pl.BlockSpec((D, K), lambda i: (0, 0)),
    ],
    out_specs=[
        pl.BlockSpec((RB, 1), lambda i: (i, 0)),
        pl.BlockSpec((1, 1), lambda i: (0, 0)),
    ],
    out_shape=[
        jax.ShapeDtypeStruct((N, 1), jnp.int32),
        jax.ShapeDtypeStruct((1, 1), jnp.float32),
    ],
    scratch_shapes=[
        pltpu.VMEM((1, K), jnp.float32),
        pltpu.VMEM((1, K), jnp.float32),
        pltpu.VMEM((1, 1), jnp.float32),
    ],
    compiler_params=pltpu.CompilerParams(
        dimension_semantics=("arbitrary",),
    ),
)


@functools.cache
def _make_gather():
    @functools.partial(
        pl.kernel,
        mesh=plsc.VectorSubcoreMesh(
            core_axis_name="c", subcore_axis_name="s", num_cores=NC
        ),
        out_type=jax.ShapeDtypeStruct((N, D), jnp.float32),
        scratch_types=[
            pltpu.VMEM((CH,), jnp.int32),
            pltpu.VMEM((CH, D), jnp.float32),
            pltpu.SemaphoreType.DMA,
        ],
    )
    def _gather(table_hbm, idx_hbm, out_hbm, idx_v, rows_v, sem):
        wid = lax.axis_index("s") * NC + lax.axis_index("c")
        for c in range(NCHUNK):
            base = wid * BPW + c * CH
            pltpu.sync_copy(idx_hbm.at[pl.ds(base, CH)], idx_v)
            pltpu.async_copy(table_hbm.at[idx_v], rows_v, sem).wait()
            pltpu.sync_copy(rows_v, out_hbm.at[pl.ds(base, CH)])

    return _gather


def kernel(x, vectors):
    idx2d, loss = _argmin_call(x, vectors)
    quantized = _make_gather()(vectors.T, idx2d.reshape(-1))
    l = loss[0, 0]
    return quantized, l, l, idx2d
